# initial kernel scaffold (unmeasured)
import jax
import jax.numpy as jnp
from jax import lax
from jax.experimental import pallas as pl
from jax.experimental.pallas import tpu as pltpu

N_DEV = 4
SQ = 1024
SKV_SHARD = 1024
HQ = 8
DH = 128
D = 1024
BLK = 64
SCALE = 0.08838834764831843


def kernel(x, Wq, K_ext, V_ext, Wo):
    def body(x_ref, wq_ref, k_ref, v_ref, wo_ref, out_ref,
             q_buf, ctx_comm, stats_comm,
             ctx_send_sems, ctx_recv_sems, st_send_sems, st_recv_sems):
        my = lax.axis_index("i")
        left = lax.rem(my + N_DEV - 1, N_DEV)
        right = lax.rem(my + 1, N_DEV)

        barrier_sem = pltpu.get_barrier_semaphore()
        for nbr in (left, right):
            pl.semaphore_signal(
                barrier_sem, inc=1,
                device_id=(nbr,), device_id_type=pl.DeviceIdType.MESH,
            )
        pl.semaphore_wait(barrier_sem, 2)

        xb = x_ref[0].astype(jnp.bfloat16)
        wqb = wq_ref[...].astype(jnp.bfloat16)
        q = lax.dot_general(xb, wqb, (((1,), (0,)), ((), ())),
                            preferred_element_type=jnp.float32)
        q_buf[...] = q.astype(jnp.bfloat16)

        row_blk = lax.broadcasted_iota(jnp.int32, (SQ, SKV_SHARD), 0) // BLK
        col_blk = (lax.broadcasted_iota(jnp.int32, (SQ, SKV_SHARD), 1) // BLK
                   + my * (SKV_SHARD // BLK))
        mask = ((row_blk == col_blk) | (col_blk == 0)
                | (lax.rem(row_blk + col_blk, 3) == 0))

        for h in range(HQ):
            qh = q_buf[:, h * DH:(h + 1) * DH]
            kh = k_ref[0, :, h, :].astype(jnp.bfloat16)
            s = lax.dot_general(qh, kh, (((1,), (1,)), ((), ())),
                                preferred_element_type=jnp.float32) * SCALE
            s = jnp.where(mask, s, -1e9)
            m = jnp.max(s, axis=1, keepdims=True)
            w = jnp.exp(s - m)
            l = jnp.sum(w, axis=1, keepdims=True)
            vh = v_ref[0, :, h, :].astype(jnp.bfloat16)
            ctx = lax.dot_general(w.astype(jnp.bfloat16), vh,
                                  (((1,), (0,)), ((), ())),
                                  preferred_element_type=jnp.float32)
            ctx_comm[0, :, h * DH:(h + 1) * DH] = ctx.astype(jnp.bfloat16)
            stats_comm[0, :, h:h + 1] = m
            stats_comm[0, :, HQ + h:HQ + h + 1] = l

        for hop in range(N_DEV - 1):
            c_rdma = pltpu.make_async_remote_copy(
                src_ref=ctx_comm.at[hop], dst_ref=ctx_comm.at[hop + 1],
                send_sem=ctx_send_sems.at[hop], recv_sem=ctx_recv_sems.at[hop],
                device_id=(right,), device_id_type=pl.DeviceIdType.MESH,
            )
            s_rdma = pltpu.make_async_remote_copy(
                src_ref=stats_comm.at[hop], dst_ref=stats_comm.at[hop + 1],
                send_sem=st_send_sems.at[hop], recv_sem=st_recv_sems.at[hop],
                device_id=(right,), device_id_type=pl.DeviceIdType.MESH,
            )
            c_rdma.start()
            s_rdma.start()
            c_rdma.wait()
            s_rdma.wait()

        m_all = [stats_comm[s, :, 0:HQ] for s in range(N_DEV)]
        m_g = m_all[0]
        for s in range(1, N_DEV):
            m_g = jnp.maximum(m_g, m_all[s])
        scales = [jnp.exp(m_all[s] - m_g) for s in range(N_DEV)]
        den = jnp.zeros((SQ, HQ), jnp.float32)
        for s in range(N_DEV):
            den = den + stats_comm[s, :, HQ:2 * HQ] * scales[s]

        for h in range(HQ):
            num = jnp.zeros((SQ, DH), jnp.float32)
            for s in range(N_DEV):
                num = num + (ctx_comm[s, :, h * DH:(h + 1) * DH]
                             .astype(jnp.float32) * scales[s][:, h:h + 1])
            ctxh = num / den[:, h:h + 1]
            q_buf[:, h * DH:(h + 1) * DH] = ctxh.astype(jnp.bfloat16)

        out = lax.dot_general(q_buf[...], wo_ref[...].astype(jnp.bfloat16),
                              (((1,), (0,)), ((), ())),
                              preferred_element_type=jnp.float32)
        out_ref[0] = out

    return pl.pallas_call(
        body,
        out_shape=jax.ShapeDtypeStruct((1, SQ, D), jnp.float32),
        in_specs=[pl.BlockSpec(memory_space=pltpu.VMEM)] * 5,
        out_specs=pl.BlockSpec(memory_space=pltpu.VMEM),
        scratch_shapes=[
            pltpu.VMEM((SQ, D), jnp.bfloat16),
            pltpu.VMEM((N_DEV, SQ, D), jnp.bfloat16),
            pltpu.VMEM((N_DEV, SQ, 2 * HQ), jnp.float32),
            pltpu.SemaphoreType.DMA((N_DEV - 1,)),
            pltpu.SemaphoreType.DMA((N_DEV - 1,)),
            pltpu.SemaphoreType.DMA((N_DEV - 1,)),
            pltpu.SemaphoreType.DMA((N_DEV - 1,)),
        ],
        compiler_params=pltpu.CompilerParams(collective_id=0),
    )(x, Wq, K_ext, V_ext, Wo)


# baseline (device time: 174683 ns/iter reference)
import jax
import jax.numpy as jnp
from jax import lax
from jax.experimental import pallas as pl
from jax.experimental.pallas import tpu as pltpu

N_DEV = 4
SQ = 1024
SKV_SHARD = 1024
HQ = 8
DH = 128
D = 1024
BLK = 64
SCALE = 0.08838834764831843
TQ = 256
NT = SQ // TQ


def kernel(x, Wq, K_ext, V_ext, Wo):
    def body(x_ref, wq_ref, k_ref, v_ref, wo_ref, out_ref,
             q_buf, wb_buf, kb_buf, vb_buf, stage_x, stage_k,
             ctx_comm, stats_comm, load_sem,
             ctx_send_sems, ctx_recv_sems, st_send_sems, st_recv_sems):
        my = lax.axis_index("i")
        left = lax.rem(my + N_DEV - 1, N_DEV)
        right = lax.rem(my + 1, N_DEV)

        barrier_sem = pltpu.get_barrier_semaphore()
        for nbr in (left, right):
            pl.semaphore_signal(
                barrier_sem, inc=1,
                device_id=(nbr,), device_id_type=pl.DeviceIdType.MESH,
            )
        pl.semaphore_wait(barrier_sem, 2)

        def load(src, dst):
            cp = pltpu.make_async_copy(src, dst, load_sem)
            cp.start()
            cp.wait()

        load(x_ref.at[0], stage_x)
        q_buf[...] = stage_x[...].astype(jnp.bfloat16)
        load(wq_ref, stage_x)
        wb_buf[...] = stage_x[...].astype(jnp.bfloat16)
        load(k_ref.at[0], stage_k)
        kb_buf[...] = stage_k[...].astype(jnp.bfloat16)
        load(v_ref.at[0], stage_k)
        vb_buf[...] = stage_k[...].astype(jnp.bfloat16)

        def q_tile(t, carry):
            r = pl.ds(t * TQ, TQ)
            q = lax.dot_general(q_buf[r, :], wb_buf[...],
                                (((1,), (0,)), ((), ())),
                                preferred_element_type=jnp.float32)
            stage_x[r, :] = q
            return carry
        lax.fori_loop(0, NT, q_tile, 0)
        q_buf[...] = stage_x[...].astype(jnp.bfloat16)

        def attn_tile(t, carry):
            r = pl.ds(t * TQ, TQ)
            row_blk = (lax.broadcasted_iota(jnp.int32, (TQ, SKV_SHARD), 0)
                       // BLK + t * (TQ // BLK))
            col_blk = (lax.broadcasted_iota(jnp.int32, (TQ, SKV_SHARD), 1)
                       // BLK + my * (SKV_SHARD // BLK))
            mask = ((row_blk == col_blk) | (col_blk == 0)
                    | (lax.rem(row_blk + col_blk, 3) == 0))
            for h in range(HQ):
                c = pl.ds(h * DH, DH)
                s = lax.dot_general(q_buf[r, c], kb_buf[:, h, :],
                                    (((1,), (1,)), ((), ())),
                                    preferred_element_type=jnp.float32) * SCALE
                s = jnp.where(mask, s, -1e9)
                m = jnp.max(s, axis=1, keepdims=True)
                w = jnp.exp(s - m)
                l = jnp.sum(w, axis=1, keepdims=True)
                ctx = lax.dot_general(w.astype(jnp.bfloat16), vb_buf[:, h, :],
                                      (((1,), (0,)), ((), ())),
                                      preferred_element_type=jnp.float32)
                ctx_comm[0, r, c] = ctx.astype(jnp.bfloat16)
                stats_comm[0, r, h:h + 1] = m
                stats_comm[0, r, HQ + h:HQ + h + 1] = l
            return carry
        lax.fori_loop(0, NT, attn_tile, 0)

        for hop in range(N_DEV - 1):
            c_rdma = pltpu.make_async_remote_copy(
                src_ref=ctx_comm.at[hop], dst_ref=ctx_comm.at[hop + 1],
                send_sem=ctx_send_sems.at[hop], recv_sem=ctx_recv_sems.at[hop],
                device_id=(right,), device_id_type=pl.DeviceIdType.MESH,
            )
            s_rdma = pltpu.make_async_remote_copy(
                src_ref=stats_comm.at[hop], dst_ref=stats_comm.at[hop + 1],
                send_sem=st_send_sems.at[hop], recv_sem=st_recv_sems.at[hop],
                device_id=(right,), device_id_type=pl.DeviceIdType.MESH,
            )
            c_rdma.start()
            s_rdma.start()
            c_rdma.wait()
            s_rdma.wait()

        load(wo_ref, stage_x)
        wb_buf[...] = stage_x[...].astype(jnp.bfloat16)

        def comb_tile(t, carry):
            r = pl.ds(t * TQ, TQ)
            m_all = [stats_comm[s, r, 0:HQ] for s in range(N_DEV)]
            m_g = m_all[0]
            for s in range(1, N_DEV):
                m_g = jnp.maximum(m_g, m_all[s])
            scales = [jnp.exp(m_all[s] - m_g) for s in range(N_DEV)]
            den = jnp.zeros((TQ, HQ), jnp.float32)
            for s in range(N_DEV):
                den = den + stats_comm[s, r, HQ:2 * HQ] * scales[s]
            inv_den = 1.0 / den
            for h in range(HQ):
                c = pl.ds(h * DH, DH)
                num = jnp.zeros((TQ, DH), jnp.float32)
                for s in range(N_DEV):
                    num = num + (ctx_comm[s, r, c].astype(jnp.float32)
                                 * scales[s][:, h:h + 1])
                q_buf[r, c] = (num * inv_den[:, h:h + 1]).astype(jnp.bfloat16)
            return carry
        lax.fori_loop(0, NT, comb_tile, 0)

        def out_tile(t, carry):
            r = pl.ds(t * TQ, TQ)
            out_ref[0, r, :] = lax.dot_general(
                q_buf[r, :], wb_buf[...], (((1,), (0,)), ((), ())),
                preferred_element_type=jnp.float32)
            return carry
        lax.fori_loop(0, NT, out_tile, 0)

    return pl.pallas_call(
        body,
        out_shape=jax.ShapeDtypeStruct((1, SQ, D), jnp.float32),
        in_specs=[pl.BlockSpec(memory_space=pl.ANY)] * 5,
        out_specs=pl.BlockSpec(memory_space=pltpu.VMEM),
        scratch_shapes=[
            pltpu.VMEM((SQ, D), jnp.bfloat16),
            pltpu.VMEM((D, D), jnp.bfloat16),
            pltpu.VMEM((SKV_SHARD, HQ, DH), jnp.bfloat16),
            pltpu.VMEM((SKV_SHARD, HQ, DH), jnp.bfloat16),
            pltpu.VMEM((SQ, D), jnp.float32),
            pltpu.VMEM((SKV_SHARD, HQ, DH), jnp.float32),
            pltpu.VMEM((N_DEV, SQ, D), jnp.bfloat16),
            pltpu.VMEM((N_DEV, SQ, 2 * HQ), jnp.float32),
            pltpu.SemaphoreType.DMA,
            pltpu.SemaphoreType.DMA((N_DEV - 1,)),
            pltpu.SemaphoreType.DMA((N_DEV - 1,)),
            pltpu.SemaphoreType.DMA((N_DEV - 1,)),
            pltpu.SemaphoreType.DMA((N_DEV - 1,)),
        ],
        compiler_params=pltpu.CompilerParams(
            collective_id=0, vmem_limit_bytes=100 * 1024 * 1024,
        ),
    )(x, Wq, K_ext, V_ext, Wo)


# device time: 101495 ns/iter; 1.7211x vs baseline; 1.7211x over previous
import jax
import jax.numpy as jnp
from jax import lax
from jax.experimental import pallas as pl
from jax.experimental.pallas import tpu as pltpu

N_DEV = 4
SQ = 1024
SKV_SHARD = 1024
HQ = 8
DH = 128
D = 1024
BLK = 64
SCALE = 0.08838834764831843
QT = SQ // N_DEV
NT = SQ // QT


def kernel(x, Wq, K_ext, V_ext, Wo):
    def body(x_ref, wq_ref, k_ref, v_ref, wo_ref, out_ref,
             q_buf, wb_buf, kb_buf, vb_buf, stage_x, stage_k, stage_v,
             ctx_loc, stats_loc, ctx_recv, stats_recv, out_send, out_recv,
             load_sems,
             ctx_send_sems, ctx_recv_sems, st_send_sems, st_recv_sems,
             out_send_sems, out_recv_sems):
        my = lax.axis_index("i")
        bf16 = jnp.bfloat16

        barrier_sem = pltpu.get_barrier_semaphore()
        for k in range(1, N_DEV):
            pl.semaphore_signal(
                barrier_sem, inc=1,
                device_id=(lax.rem(my + k, N_DEV),),
                device_id_type=pl.DeviceIdType.MESH,
            )
        pl.semaphore_wait(barrier_sem, N_DEV - 1)

        cp_x = pltpu.make_async_copy(x_ref.at[0], stage_x, load_sems.at[0])
        cp_k = pltpu.make_async_copy(k_ref.at[0], stage_k, load_sems.at[1])
        cp_v = pltpu.make_async_copy(v_ref.at[0], stage_v, load_sems.at[2])
        cp_x.start()
        cp_k.start()
        cp_v.start()
        cp_x.wait()
        q_buf[...] = stage_x[...].astype(bf16)
        cp_wq = pltpu.make_async_copy(wq_ref, stage_x, load_sems.at[3])
        cp_wq.start()
        cp_k.wait()
        kb_buf[...] = stage_k[...].astype(bf16)
        cp_v.wait()
        vb_buf[...] = stage_v[...].astype(bf16)
        cp_wq.wait()
        wb_buf[...] = stage_x[...].astype(bf16)

        def q_tile(t, carry):
            r = pl.ds(t * QT, QT)
            stage_x[r, :] = lax.dot_general(
                q_buf[r, :], wb_buf[...], (((1,), (0,)), ((), ())),
                preferred_element_type=jnp.float32)
            return carry
        lax.fori_loop(0, NT, q_tile, 0)
        q_buf[...] = stage_x[...].astype(bf16)

        def attn_tile(t, carry):
            r = pl.ds(t * QT, QT)
            row_blk = (lax.broadcasted_iota(jnp.int32, (QT, SKV_SHARD), 0)
                       // BLK + t * (QT // BLK))
            col_blk = (lax.broadcasted_iota(jnp.int32, (QT, SKV_SHARD), 1)
                       // BLK + my * (SKV_SHARD // BLK))
            mask = ((row_blk == col_blk) | (col_blk == 0)
                    | (lax.rem(row_blk + col_blk, 3) == 0))
            for h in range(HQ):
                c = pl.ds(h * DH, DH)
                s = lax.dot_general(q_buf[r, c], kb_buf[:, h, :],
                                    (((1,), (1,)), ((), ())),
                                    preferred_element_type=jnp.float32) * SCALE
                s = jnp.where(mask, s, -1e9)
                m = jnp.max(s, axis=1, keepdims=True)
                w = jnp.exp(s - m)
                l = jnp.sum(w, axis=1, keepdims=True)
                ctx = lax.dot_general(w.astype(bf16), vb_buf[:, h, :],
                                      (((1,), (0,)), ((), ())),
                                      preferred_element_type=jnp.float32)
                ctx_loc[r, c] = ctx.astype(bf16)
                stats_loc[r, h:h + 1] = m
                stats_loc[r, HQ + h:HQ + h + 1] = l
            return carry
        lax.fori_loop(0, NT, attn_tile, 0)

        sends = []
        for i in range(N_DEV - 1):
            peer = lax.rem(my + 1 + i, N_DEV)
            slot = (N_DEV - 2) - i
            rows = pl.ds(peer * QT, QT)
            c_rdma = pltpu.make_async_remote_copy(
                src_ref=ctx_loc.at[rows], dst_ref=ctx_recv.at[slot],
                send_sem=ctx_send_sems.at[i], recv_sem=ctx_recv_sems.at[slot],
                device_id=(peer,), device_id_type=pl.DeviceIdType.MESH,
            )
            s_rdma = pltpu.make_async_remote_copy(
                src_ref=stats_loc.at[rows], dst_ref=stats_recv.at[slot],
                send_sem=st_send_sems.at[i], recv_sem=st_recv_sems.at[slot],
                device_id=(peer,), device_id_type=pl.DeviceIdType.MESH,
            )
            c_rdma.start()
            s_rdma.start()
            sends.append(c_rdma)
            sends.append(s_rdma)

        for k in range(N_DEV - 1):
            pltpu.make_async_remote_copy(
                src_ref=ctx_recv.at[k], dst_ref=ctx_recv.at[k],
                send_sem=ctx_send_sems.at[k], recv_sem=ctx_recv_sems.at[k],
                device_id=(my,), device_id_type=pl.DeviceIdType.MESH,
            ).wait_recv()
            pltpu.make_async_remote_copy(
                src_ref=stats_recv.at[k], dst_ref=stats_recv.at[k],
                send_sem=st_send_sems.at[k], recv_sem=st_recv_sems.at[k],
                device_id=(my,), device_id_type=pl.DeviceIdType.MESH,
            ).wait_recv()

        rmy = pl.ds(my * QT, QT)
        m_all = [stats_loc[rmy, 0:HQ]] + [
            stats_recv[k, :, 0:HQ] for k in range(N_DEV - 1)]
        l_all = [stats_loc[rmy, HQ:2 * HQ]] + [
            stats_recv[k, :, HQ:2 * HQ] for k in range(N_DEV - 1)]
        m_g = m_all[0]
        for j in range(1, N_DEV):
            m_g = jnp.maximum(m_g, m_all[j])
        scales = [jnp.exp(m_all[j] - m_g) for j in range(N_DEV)]
        den = l_all[0] * scales[0]
        for j in range(1, N_DEV):
            den = den + l_all[j] * scales[j]
        inv_den = 1.0 / den
        for h in range(HQ):
            c = pl.ds(h * DH, DH)
            num = ctx_loc[rmy, c].astype(jnp.float32) * scales[0][:, h:h + 1]
            for k in range(N_DEV - 1):
                num = num + (ctx_recv[k, :, c].astype(jnp.float32)
                             * scales[k + 1][:, h:h + 1])
            q_buf[0:QT, c] = (num * inv_den[:, h:h + 1]).astype(bf16)

        cp_wo = pltpu.make_async_copy(wo_ref, stage_x, load_sems.at[0])
        cp_wo.start()
        cp_wo.wait()
        wb_buf[...] = stage_x[...].astype(bf16)
        out_q = lax.dot_general(q_buf[0:QT, :], wb_buf[...],
                                (((1,), (0,)), ((), ())),
                                preferred_element_type=jnp.float32)
        out_ref[0, rmy, :] = out_q
        out_send[...] = out_q.astype(bf16)

        for i in range(N_DEV - 1):
            peer = lax.rem(my + 1 + i, N_DEV)
            slot = (N_DEV - 2) - i
            o_rdma = pltpu.make_async_remote_copy(
                src_ref=out_send, dst_ref=out_recv.at[slot],
                send_sem=out_send_sems.at[i], recv_sem=out_recv_sems.at[slot],
                device_id=(peer,), device_id_type=pl.DeviceIdType.MESH,
            )
            o_rdma.start()
            sends.append(o_rdma)
        for k in range(N_DEV - 1):
            pltpu.make_async_remote_copy(
                src_ref=out_recv.at[k], dst_ref=out_recv.at[k],
                send_sem=out_send_sems.at[k], recv_sem=out_recv_sems.at[k],
                device_id=(my,), device_id_type=pl.DeviceIdType.MESH,
            ).wait_recv()
            origin = lax.rem(my + 1 + k, N_DEV)
            out_ref[0, pl.ds(origin * QT, QT), :] = (
                out_recv[k].astype(jnp.float32))

        for rdma in sends:
            rdma.wait_send()

    return pl.pallas_call(
        body,
        out_shape=jax.ShapeDtypeStruct((1, SQ, D), jnp.float32),
        in_specs=[pl.BlockSpec(memory_space=pl.ANY)] * 5,
        out_specs=pl.BlockSpec(memory_space=pltpu.VMEM),
        scratch_shapes=[
            pltpu.VMEM((SQ, D), jnp.bfloat16),
            pltpu.VMEM((D, D), jnp.bfloat16),
            pltpu.VMEM((SKV_SHARD, HQ, DH), jnp.bfloat16),
            pltpu.VMEM((SKV_SHARD, HQ, DH), jnp.bfloat16),
            pltpu.VMEM((SQ, D), jnp.float32),
            pltpu.VMEM((SKV_SHARD, HQ, DH), jnp.float32),
            pltpu.VMEM((SKV_SHARD, HQ, DH), jnp.float32),
            pltpu.VMEM((SQ, D), jnp.bfloat16),
            pltpu.VMEM((SQ, 2 * HQ), jnp.float32),
            pltpu.VMEM((N_DEV - 1, QT, D), jnp.bfloat16),
            pltpu.VMEM((N_DEV - 1, QT, 2 * HQ), jnp.float32),
            pltpu.VMEM((QT, D), jnp.bfloat16),
            pltpu.VMEM((N_DEV - 1, QT, D), jnp.bfloat16),
            pltpu.SemaphoreType.DMA((4,)),
            pltpu.SemaphoreType.DMA((N_DEV - 1,)),
            pltpu.SemaphoreType.DMA((N_DEV - 1,)),
            pltpu.SemaphoreType.DMA((N_DEV - 1,)),
            pltpu.SemaphoreType.DMA((N_DEV - 1,)),
            pltpu.SemaphoreType.DMA((N_DEV - 1,)),
            pltpu.SemaphoreType.DMA((N_DEV - 1,)),
        ],
        compiler_params=pltpu.CompilerParams(
            collective_id=0, vmem_limit_bytes=100 * 1024 * 1024,
        ),
    )(x, Wq, K_ext, V_ext, Wo)


# device time: 85505 ns/iter; 2.0430x vs baseline; 1.1870x over previous
import jax
import jax.numpy as jnp
from jax import lax
from jax.experimental import pallas as pl
from jax.experimental.pallas import tpu as pltpu

N_DEV = 4
SQ = 1024
SKV_SHARD = 1024
HQ = 8
DH = 128
D = 1024
BLK = 64
SCALE = 0.08838834764831843
QT = SQ // N_DEV
NT = SQ // QT


def kernel(x, Wq, K_ext, V_ext, Wo):
    def body(x_ref, wq_ref, k_ref, v_ref, wo_ref, out_ref,
             q_buf, wb_buf, kb_buf, vb_buf, stage_x, stage_k, stage_v,
             ctx_loc, stats_loc, ctx_recv, stats_recv, out_send, out_recv,
             load_sems,
             ctx_send_sems, ctx_recv_sems, st_send_sems, st_recv_sems,
             out_send_sems, out_recv_sems):
        my = lax.axis_index("i")
        bf16 = jnp.bfloat16

        barrier_sem = pltpu.get_barrier_semaphore()
        for k in range(1, N_DEV):
            pl.semaphore_signal(
                barrier_sem, inc=1,
                device_id=(lax.rem(my + k, N_DEV),),
                device_id_type=pl.DeviceIdType.MESH,
            )
        pl.semaphore_wait(barrier_sem, N_DEV - 1)

        cp_x = pltpu.make_async_copy(x_ref.at[0], stage_x, load_sems.at[0])
        cp_k = pltpu.make_async_copy(k_ref.at[0], stage_k, load_sems.at[1])
        cp_v = pltpu.make_async_copy(v_ref.at[0], stage_v, load_sems.at[2])
        cp_x.start()
        cp_k.start()
        cp_v.start()
        cp_x.wait()
        q_buf[...] = stage_x[...].astype(bf16)
        cp_wq = pltpu.make_async_copy(wq_ref, stage_x, load_sems.at[3])
        cp_wq.start()
        cp_k.wait()
        kb_buf[...] = stage_k[...].astype(bf16)
        cp_v.wait()
        vb_buf[...] = stage_v[...].astype(bf16)
        cp_wq.wait()
        wb_buf[...] = stage_x[...].astype(bf16)

        def q_tile(t, carry):
            r = pl.ds(t * QT, QT)
            stage_x[r, :] = lax.dot_general(
                q_buf[r, :], wb_buf[...], (((1,), (0,)), ((), ())),
                preferred_element_type=jnp.float32)
            return carry
        lax.fori_loop(0, NT, q_tile, 0)
        q_buf[...] = stage_x[...].astype(bf16)

        def attn_tile(t, carry):
            tile = lax.rem(my + 1 + t, N_DEV)
            r = pl.ds(tile * QT, QT)
            row_blk = (lax.broadcasted_iota(jnp.int32, (QT, SKV_SHARD), 0)
                       // BLK + tile * (QT // BLK))
            col_blk = (lax.broadcasted_iota(jnp.int32, (QT, SKV_SHARD), 1)
                       // BLK + my * (SKV_SHARD // BLK))
            mask = ((row_blk == col_blk) | (col_blk == 0)
                    | (lax.rem(row_blk + col_blk, 3) == 0))
            for h in range(HQ):
                c = pl.ds(h * DH, DH)
                s = lax.dot_general(q_buf[r, c], kb_buf[:, h, :],
                                    (((1,), (1,)), ((), ())),
                                    preferred_element_type=jnp.float32) * SCALE
                s = jnp.where(mask, s, -1e9)
                m = jnp.max(s, axis=1, keepdims=True)
                w = jnp.exp(s - m)
                l = jnp.sum(w, axis=1, keepdims=True)
                ctx = lax.dot_general(w.astype(bf16), vb_buf[:, h, :],
                                      (((1,), (0,)), ((), ())),
                                      preferred_element_type=jnp.float32)
                ctx_loc[r, c] = ctx.astype(bf16)
                stats_loc[r, h:h + 1] = m
                stats_loc[r, HQ + h:HQ + h + 1] = l

            @pl.when(t < N_DEV - 1)
            def _():
                slot = (N_DEV - 2) - t
                pltpu.make_async_remote_copy(
                    src_ref=ctx_loc.at[r], dst_ref=ctx_recv.at[slot],
                    send_sem=ctx_send_sems.at[t],
                    recv_sem=ctx_recv_sems.at[slot],
                    device_id=(tile,), device_id_type=pl.DeviceIdType.MESH,
                ).start()
                pltpu.make_async_remote_copy(
                    src_ref=stats_loc.at[r], dst_ref=stats_recv.at[slot],
                    send_sem=st_send_sems.at[t],
                    recv_sem=st_recv_sems.at[slot],
                    device_id=(tile,), device_id_type=pl.DeviceIdType.MESH,
                ).start()
            return carry
        lax.fori_loop(0, NT, attn_tile, 0)

        sends = []
        for i in range(N_DEV - 1):
            sends.append(pltpu.make_async_remote_copy(
                src_ref=ctx_loc.at[pl.ds(0, QT)], dst_ref=ctx_recv.at[0],
                send_sem=ctx_send_sems.at[i], recv_sem=ctx_recv_sems.at[0],
                device_id=(my,), device_id_type=pl.DeviceIdType.MESH,
            ))
            sends.append(pltpu.make_async_remote_copy(
                src_ref=stats_loc.at[pl.ds(0, QT)], dst_ref=stats_recv.at[0],
                send_sem=st_send_sems.at[i], recv_sem=st_recv_sems.at[0],
                device_id=(my,), device_id_type=pl.DeviceIdType.MESH,
            ))

        for k in range(N_DEV - 1):
            pltpu.make_async_remote_copy(
                src_ref=ctx_recv.at[k], dst_ref=ctx_recv.at[k],
                send_sem=ctx_send_sems.at[k], recv_sem=ctx_recv_sems.at[k],
                device_id=(my,), device_id_type=pl.DeviceIdType.MESH,
            ).wait_recv()
            pltpu.make_async_remote_copy(
                src_ref=stats_recv.at[k], dst_ref=stats_recv.at[k],
                send_sem=st_send_sems.at[k], recv_sem=st_recv_sems.at[k],
                device_id=(my,), device_id_type=pl.DeviceIdType.MESH,
            ).wait_recv()

        rmy = pl.ds(my * QT, QT)
        m_all = [stats_loc[rmy, 0:HQ]] + [
            stats_recv[k, :, 0:HQ] for k in range(N_DEV - 1)]
        l_all = [stats_loc[rmy, HQ:2 * HQ]] + [
            stats_recv[k, :, HQ:2 * HQ] for k in range(N_DEV - 1)]
        m_g = m_all[0]
        for j in range(1, N_DEV):
            m_g = jnp.maximum(m_g, m_all[j])
        scales = [jnp.exp(m_all[j] - m_g) for j in range(N_DEV)]
        den = l_all[0] * scales[0]
        for j in range(1, N_DEV):
            den = den + l_all[j] * scales[j]
        inv_den = 1.0 / den
        for h in range(HQ):
            c = pl.ds(h * DH, DH)
            num = ctx_loc[rmy, c].astype(jnp.float32) * scales[0][:, h:h + 1]
            for k in range(N_DEV - 1):
                num = num + (ctx_recv[k, :, c].astype(jnp.float32)
                             * scales[k + 1][:, h:h + 1])
            q_buf[0:QT, c] = (num * inv_den[:, h:h + 1]).astype(bf16)

        cp_wo = pltpu.make_async_copy(wo_ref, stage_x, load_sems.at[0])
        cp_wo.start()
        cp_wo.wait()
        wb_buf[...] = stage_x[...].astype(bf16)
        out_q = lax.dot_general(q_buf[0:QT, :], wb_buf[...],
                                (((1,), (0,)), ((), ())),
                                preferred_element_type=jnp.float32)
        out_ref[0, rmy, :] = out_q
        out_send[...] = out_q.astype(bf16)

        for i in range(N_DEV - 1):
            peer = lax.rem(my + 1 + i, N_DEV)
            slot = (N_DEV - 2) - i
            o_rdma = pltpu.make_async_remote_copy(
                src_ref=out_send, dst_ref=out_recv.at[slot],
                send_sem=out_send_sems.at[i], recv_sem=out_recv_sems.at[slot],
                device_id=(peer,), device_id_type=pl.DeviceIdType.MESH,
            )
            o_rdma.start()
            sends.append(o_rdma)
        for k in range(N_DEV - 1):
            pltpu.make_async_remote_copy(
                src_ref=out_recv.at[k], dst_ref=out_recv.at[k],
                send_sem=out_send_sems.at[k], recv_sem=out_recv_sems.at[k],
                device_id=(my,), device_id_type=pl.DeviceIdType.MESH,
            ).wait_recv()
            origin = lax.rem(my + 1 + k, N_DEV)
            out_ref[0, pl.ds(origin * QT, QT), :] = (
                out_recv[k].astype(jnp.float32))

        for rdma in sends:
            rdma.wait_send()

    return pl.pallas_call(
        body,
        out_shape=jax.ShapeDtypeStruct((1, SQ, D), jnp.float32),
        in_specs=[pl.BlockSpec(memory_space=pl.ANY)] * 5,
        out_specs=pl.BlockSpec(memory_space=pltpu.VMEM),
        scratch_shapes=[
            pltpu.VMEM((SQ, D), jnp.bfloat16),
            pltpu.VMEM((D, D), jnp.bfloat16),
            pltpu.VMEM((SKV_SHARD, HQ, DH), jnp.bfloat16),
            pltpu.VMEM((SKV_SHARD, HQ, DH), jnp.bfloat16),
            pltpu.VMEM((SQ, D), jnp.float32),
            pltpu.VMEM((SKV_SHARD, HQ, DH), jnp.float32),
            pltpu.VMEM((SKV_SHARD, HQ, DH), jnp.float32),
            pltpu.VMEM((SQ, D), jnp.bfloat16),
            pltpu.VMEM((SQ, 2 * HQ), jnp.float32),
            pltpu.VMEM((N_DEV - 1, QT, D), jnp.bfloat16),
            pltpu.VMEM((N_DEV - 1, QT, 2 * HQ), jnp.float32),
            pltpu.VMEM((QT, D), jnp.bfloat16),
            pltpu.VMEM((N_DEV - 1, QT, D), jnp.bfloat16),
            pltpu.SemaphoreType.DMA((4,)),
            pltpu.SemaphoreType.DMA((N_DEV - 1,)),
            pltpu.SemaphoreType.DMA((N_DEV - 1,)),
            pltpu.SemaphoreType.DMA((N_DEV - 1,)),
            pltpu.SemaphoreType.DMA((N_DEV - 1,)),
            pltpu.SemaphoreType.DMA((N_DEV - 1,)),
            pltpu.SemaphoreType.DMA((N_DEV - 1,)),
        ],
        compiler_params=pltpu.CompilerParams(
            collective_id=0, vmem_limit_bytes=100 * 1024 * 1024,
        ),
    )(x, Wq, K_ext, V_ext, Wo)


# device time: 85248 ns/iter; 2.0491x vs baseline; 1.0030x over previous
import jax
import jax.numpy as jnp
from jax import lax
from jax.experimental import pallas as pl
from jax.experimental.pallas import tpu as pltpu

N_DEV = 4
SQ = 1024
SKV_SHARD = 1024
HQ = 8
DH = 128
D = 1024
BLK = 64
SCALE = 0.08838834764831843
QT = SQ // N_DEV
NT = SQ // QT


def kernel(x, Wq, K_ext, V_ext, Wo):
    def body(x_ref, wq_ref, k_ref, v_ref, wo_ref, out_ref,
             q_buf, wb_buf, kb_buf, vb_buf, stage_x, stage_k, stage_v,
             ctx_loc, stats_loc, ctx_recv, stats_recv, out_send, out_recv,
             load_sems,
             ctx_send_sems, ctx_recv_sems, st_send_sems, st_recv_sems,
             out_send_sems, out_recv_sems):
        my = lax.axis_index("i")
        bf16 = jnp.bfloat16

        barrier_sem = pltpu.get_barrier_semaphore()
        for k in range(1, N_DEV):
            pl.semaphore_signal(
                barrier_sem, inc=1,
                device_id=(lax.rem(my + k, N_DEV),),
                device_id_type=pl.DeviceIdType.MESH,
            )
        pl.semaphore_wait(barrier_sem, N_DEV - 1)

        cp_x = pltpu.make_async_copy(x_ref.at[0], stage_x, load_sems.at[0])
        cp_k = pltpu.make_async_copy(k_ref.at[0], stage_k, load_sems.at[1])
        cp_v = pltpu.make_async_copy(v_ref.at[0], stage_v, load_sems.at[2])
        cp_x.start()
        cp_k.start()
        cp_v.start()
        cp_x.wait()
        q_buf[...] = stage_x[...].astype(bf16)
        cp_wq = pltpu.make_async_copy(wq_ref, stage_x, load_sems.at[3])
        cp_wq.start()
        cp_k.wait()
        kb_buf[...] = stage_k[...].astype(bf16)
        cp_v.wait()
        vb_buf[...] = stage_v[...].astype(bf16)
        cp_wq.wait()
        wb_buf[...] = stage_x[...].astype(bf16)

        def q_tile(t, carry):
            r = pl.ds(t * QT, QT)
            stage_x[r, :] = lax.dot_general(
                q_buf[r, :], wb_buf[...], (((1,), (0,)), ((), ())),
                preferred_element_type=jnp.float32)
            return carry
        lax.fori_loop(0, NT, q_tile, 0)
        q_buf[...] = stage_x[...].astype(bf16)

        def attn_tile(t, carry):
            tile = lax.rem(my + 1 + t, N_DEV)
            r = pl.ds(tile * QT, QT)
            row_blk = (lax.broadcasted_iota(jnp.int32, (QT, SKV_SHARD), 0)
                       // BLK + tile * (QT // BLK))
            col_blk = (lax.broadcasted_iota(jnp.int32, (QT, SKV_SHARD), 1)
                       // BLK + my * (SKV_SHARD // BLK))
            mask = ((row_blk == col_blk) | (col_blk == 0)
                    | (lax.rem(row_blk + col_blk, 3) == 0))
            for h in range(HQ):
                c = pl.ds(h * DH, DH)
                s = lax.dot_general(q_buf[r, c], kb_buf[:, h, :],
                                    (((1,), (1,)), ((), ())),
                                    preferred_element_type=jnp.float32) * SCALE
                s = jnp.where(mask, s, -1e9)
                m = jnp.max(s, axis=1, keepdims=True)
                w = jnp.exp(s - m)
                l = jnp.sum(w, axis=1, keepdims=True)
                ctx = lax.dot_general(w.astype(bf16), vb_buf[:, h, :],
                                      (((1,), (0,)), ((), ())),
                                      preferred_element_type=jnp.float32)
                ctx_loc[r, c] = ctx.astype(bf16)
                stats_loc[r, h:h + 1] = m
                stats_loc[r, HQ + h:HQ + h + 1] = l

            @pl.when(t < N_DEV - 1)
            def _():
                slot = (N_DEV - 2) - t
                pltpu.make_async_remote_copy(
                    src_ref=ctx_loc.at[r], dst_ref=ctx_recv.at[slot],
                    send_sem=ctx_send_sems.at[t],
                    recv_sem=ctx_recv_sems.at[slot],
                    device_id=(tile,), device_id_type=pl.DeviceIdType.MESH,
                ).start()
                pltpu.make_async_remote_copy(
                    src_ref=stats_loc.at[r], dst_ref=stats_recv.at[slot],
                    send_sem=st_send_sems.at[t],
                    recv_sem=st_recv_sems.at[slot],
                    device_id=(tile,), device_id_type=pl.DeviceIdType.MESH,
                ).start()
            return carry
        lax.fori_loop(0, NT, attn_tile, 0)

        sends = []
        for i in range(N_DEV - 1):
            sends.append(pltpu.make_async_remote_copy(
                src_ref=ctx_loc.at[pl.ds(0, QT)], dst_ref=ctx_recv.at[0],
                send_sem=ctx_send_sems.at[i], recv_sem=ctx_recv_sems.at[0],
                device_id=(my,), device_id_type=pl.DeviceIdType.MESH,
            ))
            sends.append(pltpu.make_async_remote_copy(
                src_ref=stats_loc.at[pl.ds(0, QT)], dst_ref=stats_recv.at[0],
                send_sem=st_send_sems.at[i], recv_sem=st_recv_sems.at[0],
                device_id=(my,), device_id_type=pl.DeviceIdType.MESH,
            ))

        cp_wo = pltpu.make_async_copy(wo_ref, stage_x, load_sems.at[0])
        cp_wo.start()

        for k in range(N_DEV - 1):
            pltpu.make_async_remote_copy(
                src_ref=ctx_recv.at[k], dst_ref=ctx_recv.at[k],
                send_sem=ctx_send_sems.at[k], recv_sem=ctx_recv_sems.at[k],
                device_id=(my,), device_id_type=pl.DeviceIdType.MESH,
            ).wait_recv()
            pltpu.make_async_remote_copy(
                src_ref=stats_recv.at[k], dst_ref=stats_recv.at[k],
                send_sem=st_send_sems.at[k], recv_sem=st_recv_sems.at[k],
                device_id=(my,), device_id_type=pl.DeviceIdType.MESH,
            ).wait_recv()

        rmy = pl.ds(my * QT, QT)
        m_all = [stats_loc[rmy, 0:HQ]] + [
            stats_recv[k, :, 0:HQ] for k in range(N_DEV - 1)]
        l_all = [stats_loc[rmy, HQ:2 * HQ]] + [
            stats_recv[k, :, HQ:2 * HQ] for k in range(N_DEV - 1)]
        m_g = m_all[0]
        for j in range(1, N_DEV):
            m_g = jnp.maximum(m_g, m_all[j])
        scales = [jnp.exp(m_all[j] - m_g) for j in range(N_DEV)]
        den = l_all[0] * scales[0]
        for j in range(1, N_DEV):
            den = den + l_all[j] * scales[j]
        inv_den = 1.0 / den
        for h in range(HQ):
            c = pl.ds(h * DH, DH)
            num = ctx_loc[rmy, c].astype(jnp.float32) * scales[0][:, h:h + 1]
            for k in range(N_DEV - 1):
                num = num + (ctx_recv[k, :, c].astype(jnp.float32)
                             * scales[k + 1][:, h:h + 1])
            q_buf[0:QT, c] = (num * inv_den[:, h:h + 1]).astype(bf16)

        cp_wo.wait()
        wb_buf[...] = stage_x[...].astype(bf16)
        out_q = lax.dot_general(q_buf[0:QT, :], wb_buf[...],
                                (((1,), (0,)), ((), ())),
                                preferred_element_type=jnp.float32)
        out_ref[0, rmy, :] = out_q
        out_send[...] = out_q.astype(bf16)

        for i in range(N_DEV - 1):
            peer = lax.rem(my + 1 + i, N_DEV)
            slot = (N_DEV - 2) - i
            o_rdma = pltpu.make_async_remote_copy(
                src_ref=out_send, dst_ref=out_recv.at[slot],
                send_sem=out_send_sems.at[i], recv_sem=out_recv_sems.at[slot],
                device_id=(peer,), device_id_type=pl.DeviceIdType.MESH,
            )
            o_rdma.start()
            sends.append(o_rdma)
        for k in range(N_DEV - 1):
            pltpu.make_async_remote_copy(
                src_ref=out_recv.at[k], dst_ref=out_recv.at[k],
                send_sem=out_send_sems.at[k], recv_sem=out_recv_sems.at[k],
                device_id=(my,), device_id_type=pl.DeviceIdType.MESH,
            ).wait_recv()
            origin = lax.rem(my + 1 + k, N_DEV)
            out_ref[0, pl.ds(origin * QT, QT), :] = (
                out_recv[k].astype(jnp.float32))

        for rdma in sends:
            rdma.wait_send()

    return pl.pallas_call(
        body,
        out_shape=jax.ShapeDtypeStruct((1, SQ, D), jnp.float32),
        in_specs=[pl.BlockSpec(memory_space=pl.ANY)] * 5,
        out_specs=pl.BlockSpec(memory_space=pltpu.VMEM),
        scratch_shapes=[
            pltpu.VMEM((SQ, D), jnp.bfloat16),
            pltpu.VMEM((D, D), jnp.bfloat16),
            pltpu.VMEM((SKV_SHARD, HQ, DH), jnp.bfloat16),
            pltpu.VMEM((SKV_SHARD, HQ, DH), jnp.bfloat16),
            pltpu.VMEM((SQ, D), jnp.float32),
            pltpu.VMEM((SKV_SHARD, HQ, DH), jnp.float32),
            pltpu.VMEM((SKV_SHARD, HQ, DH), jnp.float32),
            pltpu.VMEM((SQ, D), jnp.bfloat16),
            pltpu.VMEM((SQ, 2 * HQ), jnp.float32),
            pltpu.VMEM((N_DEV - 1, QT, D), jnp.bfloat16),
            pltpu.VMEM((N_DEV - 1, QT, 2 * HQ), jnp.float32),
            pltpu.VMEM((QT, D), jnp.bfloat16),
            pltpu.VMEM((N_DEV - 1, QT, D), jnp.bfloat16),
            pltpu.SemaphoreType.DMA((4,)),
            pltpu.SemaphoreType.DMA((N_DEV - 1,)),
            pltpu.SemaphoreType.DMA((N_DEV - 1,)),
            pltpu.SemaphoreType.DMA((N_DEV - 1,)),
            pltpu.SemaphoreType.DMA((N_DEV - 1,)),
            pltpu.SemaphoreType.DMA((N_DEV - 1,)),
            pltpu.SemaphoreType.DMA((N_DEV - 1,)),
        ],
        compiler_params=pltpu.CompilerParams(
            collective_id=0, vmem_limit_bytes=100 * 1024 * 1024,
        ),
    )(x, Wq, K_ext, V_ext, Wo)
